# baseline (speedup 1.0000x reference)
import jax
import jax.numpy as jnp
from jax.experimental import pallas as pl
from jax.experimental.pallas import tpu as pltpu

N = 10000
BM = 640
NB = -(-N // BM)

_TDIMS = (((1,), (1,)), ((), ()))


def _gcn_kernel(adj_ref, x_ref, w1_ref, b1_ref, w2_ref, b2_ref,
                out_ref, s1t_scr, s2t_scr):
    i = pl.program_id(0)

    @pl.when(i == 0)
    def _():
        s1t_scr[...] = jax.lax.dot_general(
            w1_ref[...].T, x_ref[...], _TDIMS,
            preferred_element_type=jnp.float32)

    @pl.when(i < NB)
    def _():
        h = jax.lax.dot_general(adj_ref[...], s1t_scr[...], _TDIMS,
                                preferred_element_type=jnp.float32)
        h = jnp.maximum(h + b1_ref[...], 0.0)
        s2t_scr[:, pl.ds(i * BM, BM)] = jax.lax.dot_general(
            w2_ref[...], h, (((0,), (1,)), ((), ())),
            preferred_element_type=jnp.float32)

    @pl.when(i >= NB)
    def _():
        out_ref[...] = jax.lax.dot_general(
            adj_ref[...], s2t_scr[:, pl.ds(0, N)], _TDIMS,
            preferred_element_type=jnp.float32) + b2_ref[...]


@jax.jit
def kernel(x, adj, W1, b1, W2, b2):
    nfeat = x.shape[1]
    nhid = W1.shape[1]
    nclass = W2.shape[1]
    b1_2d = b1.reshape(1, nhid)
    b2_2d = b2.reshape(1, nclass)

    return pl.pallas_call(
        _gcn_kernel,
        grid=(2 * NB,),
        out_shape=jax.ShapeDtypeStruct((N, nclass), jnp.float32),
        in_specs=[
            pl.BlockSpec((BM, N), lambda i: (jax.lax.rem(i, NB), 0)),
            pl.BlockSpec((N, nfeat), lambda i: (0, 0)),
            pl.BlockSpec((nfeat, nhid), lambda i: (0, 0)),
            pl.BlockSpec((1, nhid), lambda i: (0, 0)),
            pl.BlockSpec((nhid, nclass), lambda i: (0, 0)),
            pl.BlockSpec((1, nclass), lambda i: (0, 0)),
        ],
        out_specs=pl.BlockSpec(
            (BM, nclass),
            lambda i: (jnp.where(i < NB, 0, i - NB), 0)),
        scratch_shapes=[
            pltpu.VMEM((nhid, N), jnp.float32),
            pltpu.VMEM((nclass, NB * BM), jnp.float32),
        ],
        compiler_params=pltpu.CompilerParams(
            dimension_semantics=("arbitrary",),
        ),
    )(adj, x, W1, b1_2d, W2, b2_2d)


# speedup vs baseline: 1.0005x; 1.0005x over previous
import jax
import jax.numpy as jnp
from jax.experimental import pallas as pl
from jax.experimental.pallas import tpu as pltpu

N = 10000
BM = 400
NB = N // BM
NSLOT = 3

_TDIMS = (((1,), (1,)), ((), ()))


def _gcn_kernel(x_ref, w1_ref, b1_ref, w2_ref, b2_ref, adj_ref, out_ref,
                bufs, s1t_scr, s2_scr, oblk, dsem, osem):
    for b in range(NSLOT):
        pltpu.make_async_copy(adj_ref.at[pl.ds(b * BM, BM), :],
                              bufs.at[b], dsem.at[b]).start()

    s1t_scr[...] = jax.lax.dot_general(
        w1_ref[...].T, x_ref[...], _TDIMS,
        preferred_element_type=jnp.float32)

    def step(i, _):
        slot = jax.lax.rem(i, NSLOT)
        blk = jnp.where(i < NB, i, i - NB)
        pltpu.make_async_copy(adj_ref.at[pl.ds(blk * BM, BM), :],
                              bufs.at[slot], dsem.at[slot]).wait()

        @pl.when(i < NB)
        def _():
            h = jax.lax.dot_general(bufs[slot], s1t_scr[...], _TDIMS,
                                    preferred_element_type=jnp.float32)
            h = jnp.maximum(h + b1_ref[...], 0.0)
            s2_scr[pl.ds(blk * BM, BM), :] = jnp.dot(
                h, w2_ref[...], preferred_element_type=jnp.float32)

        @pl.when(i >= NB)
        def _():
            oslot = jax.lax.rem(i, 2)

            @pl.when(i >= NB + 2)
            def _():
                pltpu.make_async_copy(
                    oblk.at[oslot],
                    out_ref.at[pl.ds((blk - 2) * BM, BM), :],
                    osem.at[oslot]).wait()

            oblk[oslot] = jnp.dot(
                bufs[slot], s2_scr[...],
                preferred_element_type=jnp.float32) + b2_ref[...]
            pltpu.make_async_copy(oblk.at[oslot],
                                  out_ref.at[pl.ds(blk * BM, BM), :],
                                  osem.at[oslot]).start()

        nxt = i + NSLOT

        @pl.when(nxt < 2 * NB)
        def _():
            nblk = jnp.where(nxt < NB, nxt, nxt - NB)
            pltpu.make_async_copy(adj_ref.at[pl.ds(nblk * BM, BM), :],
                                  bufs.at[slot], dsem.at[slot]).start()

        return 0

    jax.lax.fori_loop(0, 2 * NB, step, 0)

    for last in (NB - 2, NB - 1):
        pltpu.make_async_copy(
            oblk.at[(NB + last) % 2],
            out_ref.at[pl.ds(last * BM, BM), :],
            osem.at[(NB + last) % 2]).wait()


@jax.jit
def kernel(x, adj, W1, b1, W2, b2):
    nfeat = x.shape[1]
    nhid = W1.shape[1]
    nclass = W2.shape[1]

    return pl.pallas_call(
        _gcn_kernel,
        out_shape=jax.ShapeDtypeStruct((N, nclass), jnp.float32),
        in_specs=[
            pl.BlockSpec(memory_space=pltpu.VMEM),
            pl.BlockSpec(memory_space=pltpu.VMEM),
            pl.BlockSpec(memory_space=pltpu.VMEM),
            pl.BlockSpec(memory_space=pltpu.VMEM),
            pl.BlockSpec(memory_space=pltpu.VMEM),
            pl.BlockSpec(memory_space=pltpu.HBM),
        ],
        out_specs=pl.BlockSpec(memory_space=pltpu.HBM),
        scratch_shapes=[
            pltpu.VMEM((NSLOT, BM, N), jnp.float32),
            pltpu.VMEM((nhid, N), jnp.float32),
            pltpu.VMEM((N, nclass), jnp.float32),
            pltpu.VMEM((2, BM, nclass), jnp.float32),
            pltpu.SemaphoreType.DMA((NSLOT,)),
            pltpu.SemaphoreType.DMA((2,)),
        ],
    )(x, W1, b1.reshape(1, nhid), W2, b2.reshape(1, nclass), adj)


# speedup vs baseline: 1.0118x; 1.0113x over previous
import jax
import jax.numpy as jnp
from jax.experimental import pallas as pl
from jax.experimental.pallas import tpu as pltpu

N = 10000
BM = 400
NB = N // BM


def _gcn_kernel(adj_ref, x_ref, w1_ref, b1_ref, w2_ref, b2_ref,
                out_ref, s1_scr, s2_scr):
    i = pl.program_id(0)

    @pl.when(i == 0)
    def _():
        s1_scr[...] = jnp.dot(x_ref[...], w1_ref[...],
                              preferred_element_type=jnp.float32)

    @pl.when(i < NB)
    def _():
        h = jnp.dot(adj_ref[...], s1_scr[...],
                    preferred_element_type=jnp.float32)
        h = jnp.maximum(h + b1_ref[...], 0.0)
        s2_scr[pl.ds(i * BM, BM), :] = jnp.dot(
            h, w2_ref[...], preferred_element_type=jnp.float32)

    @pl.when(i >= NB)
    def _():
        out_ref[...] = jnp.dot(adj_ref[...], s2_scr[...],
                               preferred_element_type=jnp.float32) + b2_ref[...]


@jax.jit
def kernel(x, adj, W1, b1, W2, b2):
    nfeat = x.shape[1]
    nhid = W1.shape[1]
    nclass = W2.shape[1]
    b1_2d = b1.reshape(1, nhid)
    b2_2d = b2.reshape(1, nclass)

    return pl.pallas_call(
        _gcn_kernel,
        grid=(2 * NB,),
        out_shape=jax.ShapeDtypeStruct((N, nclass), jnp.float32),
        in_specs=[
            pl.BlockSpec((BM, N), lambda i: (jax.lax.rem(i, NB), 0)),
            pl.BlockSpec((N, nfeat), lambda i: (0, 0)),
            pl.BlockSpec((nfeat, nhid), lambda i: (0, 0)),
            pl.BlockSpec((1, nhid), lambda i: (0, 0)),
            pl.BlockSpec((nhid, nclass), lambda i: (0, 0)),
            pl.BlockSpec((1, nclass), lambda i: (0, 0)),
        ],
        out_specs=pl.BlockSpec(
            (BM, nclass),
            lambda i: (jnp.where(i < NB, 0, i - NB), 0)),
        scratch_shapes=[
            pltpu.VMEM((N, nhid), jnp.float32),
            pltpu.VMEM((N, nclass), jnp.float32),
        ],
        compiler_params=pltpu.CompilerParams(
            dimension_semantics=("arbitrary",),
        ),
    )(adj, x, W1, b1_2d, W2, b2_2d)
